# baseline (speedup 1.0000x reference)
import jax
import jax.numpy as jnp
from jax.experimental import pallas as pl
from jax.experimental.pallas import tpu as pltpu


def _compute(adj_buf, nodes_buf, w_ref, out_buf, i):
    a = adj_buf[i].astype(jnp.bfloat16)
    x = nodes_buf[i].astype(jnp.bfloat16)
    agg = jax.lax.dot_general(
        a, x, (((2,), (1,)), ((0,), (0,))),
        preferred_element_type=jnp.float32).astype(jnp.bfloat16)
    w = w_ref[...].astype(jnp.bfloat16)
    out_buf[i] = jax.lax.dot_general(
        agg, w, (((2,), (1,)), ((0,), (0,))),
        preferred_element_type=jnp.float32)


def _make_body(nchunks):
    def _body(adj_hbm, nodes_hbm, w_ref, out_hbm,
              adj_buf, nodes_buf, out_buf, adj_sem, nodes_sem, out_sem):
        c = pl.program_id(0)
        for i in range(nchunks):
            pltpu.make_async_copy(adj_hbm.at[c, i], adj_buf.at[i],
                                  adj_sem.at[i]).start()
            pltpu.make_async_copy(nodes_hbm.at[c, i], nodes_buf.at[i],
                                  nodes_sem.at[i]).start()
        for i in range(nchunks):
            pltpu.make_async_copy(adj_hbm.at[c, i], adj_buf.at[i],
                                  adj_sem.at[i]).wait()
            pltpu.make_async_copy(nodes_hbm.at[c, i], nodes_buf.at[i],
                                  nodes_sem.at[i]).wait()
            _compute(adj_buf, nodes_buf, w_ref, out_buf, i)
            pltpu.make_async_copy(out_buf.at[i], out_hbm.at[c, i],
                                  out_sem.at[i]).start()
        for i in range(nchunks):
            pltpu.make_async_copy(out_buf.at[i], out_hbm.at[c, i],
                                  out_sem.at[i]).wait()
    return _body


def kernel(adjacency, nodes, weights):
    adjacency = adjacency.astype(jnp.float32)
    nodes = nodes.astype(jnp.float32)
    weights = weights.astype(jnp.float32)

    B, W, N, _ = adjacency.shape
    Fin = nodes.shape[-1]
    Wp, _, Fout = weights.shape
    w_used = weights[Wp - W:, :, :]

    ncores = 2 if B % 2 == 0 else 1
    bpc = B // ncores
    cb = 2 if bpc % 2 == 0 else 1
    nchunks = bpc // cb
    G = cb * W
    adj_r = adjacency.reshape(ncores, nchunks, G, N, N)
    nodes_r = nodes.reshape(ncores, nchunks, G, N, Fin)
    w_g = jnp.broadcast_to(w_used[None], (cb, W, Fin, Fout)).reshape(
        G, Fin, Fout)

    out = pl.pallas_call(
        _make_body(nchunks),
        grid=(ncores,),
        in_specs=[
            pl.BlockSpec(memory_space=pl.ANY),
            pl.BlockSpec(memory_space=pl.ANY),
            pl.BlockSpec((G, Fin, Fout), lambda c: (0, 0, 0)),
        ],
        out_specs=pl.BlockSpec(memory_space=pl.ANY),
        out_shape=jax.ShapeDtypeStruct((ncores, nchunks, G, N, Fout),
                                       jnp.float32),
        scratch_shapes=[
            pltpu.VMEM((nchunks, G, N, N), jnp.float32),
            pltpu.VMEM((nchunks, G, N, Fin), jnp.float32),
            pltpu.VMEM((nchunks, G, N, Fout), jnp.float32),
            pltpu.SemaphoreType.DMA((nchunks,)),
            pltpu.SemaphoreType.DMA((nchunks,)),
            pltpu.SemaphoreType.DMA((nchunks,)),
        ],
        compiler_params=pltpu.CompilerParams(
            dimension_semantics=("parallel",)),
    )(adj_r, nodes_r, w_g)
    return out.reshape(B, W, N, Fout)


# speedup vs baseline: 1.2537x; 1.2537x over previous
import jax
import jax.numpy as jnp
from jax.experimental import pallas as pl
from jax.experimental.pallas import tpu as pltpu


def _gcn_body(adj_t_ref, adj_b_ref, nodes_ref, w_ref, out_ref):
    x = nodes_ref[...].astype(jnp.bfloat16)
    w = w_ref[...].astype(jnp.bfloat16)
    half = adj_t_ref.shape[1]
    for k, a_ref in ((0, adj_t_ref), (1, adj_b_ref)):
        a = a_ref[...].astype(jnp.bfloat16)
        agg = jax.lax.dot_general(
            a, x, (((2,), (1,)), ((0,), (0,))),
            preferred_element_type=jnp.float32).astype(jnp.bfloat16)
        out_ref[:, k * half:(k + 1) * half, :] = jax.lax.dot_general(
            agg, w, (((2,), (1,)), ((0,), (0,))),
            preferred_element_type=jnp.float32)


def kernel(adjacency, nodes, weights):
    adjacency = adjacency.astype(jnp.float32)
    nodes = nodes.astype(jnp.float32)
    weights = weights.astype(jnp.float32)

    B, W, N, _ = adjacency.shape
    Fin = nodes.shape[-1]
    Wp, _, Fout = weights.shape
    w_used = weights[Wp - W:, :, :]
    half = N // 2

    return pl.pallas_call(
        _gcn_body,
        grid=(B,),
        in_specs=[
            pl.BlockSpec((None, W, half, N), lambda b: (b, 0, 0, 0)),
            pl.BlockSpec((None, W, half, N), lambda b: (b, 0, 1, 0)),
            pl.BlockSpec((None, W, N, Fin), lambda b: (b, 0, 0, 0)),
            pl.BlockSpec((W, Fin, Fout), lambda b: (0, 0, 0)),
        ],
        out_specs=pl.BlockSpec((None, W, N, Fout), lambda b: (b, 0, 0, 0)),
        out_shape=jax.ShapeDtypeStruct((B, W, N, Fout), jnp.float32),
        compiler_params=pltpu.CompilerParams(
            dimension_semantics=("parallel",)),
    )(adjacency, adjacency, nodes, w_used)
